# baseline (speedup 1.0000x reference)
import functools

import jax
import jax.numpy as jnp
from jax import lax
from jax.experimental import pallas as pl
from jax.experimental.pallas import tpu as pltpu
from jax.experimental.pallas import tpu_sc as plsc

_NH = 4
_NPNT = 4
_KQ = 8
_TAPS = 16


def _proj_body(x1_ref, x2_ref, qpos_ref, ln1w_ref, ln1b_ref, ln2w_ref, ln2b_ref,
               sow_ref, sob_ref, aww_ref, awb_ref, vpw_ref, vpb_ref,
               vh_ref, tidx_ref, tw_ref):
    nq, C = x1_ref.shape[1], x1_ref.shape[2]
    W = 32
    hd = C // _NH
    b = pl.program_id(0)

    x1b = x1_ref[0]
    x2b = x2_ref[0]

    def ln(x, w, b_):
        mu = jnp.mean(x, axis=-1, keepdims=True)
        xc = x - mu
        var = jnp.mean(xc * xc, axis=-1, keepdims=True)
        return xc * lax.rsqrt(var + 1e-5) * w + b_

    query = ln(x1b, ln1w_ref[0], ln1b_ref[0]) + qpos_ref[...]
    value = ln(x2b, ln2w_ref[0], ln2b_ref[0])
    v = jnp.dot(value, vpw_ref[...], preferred_element_type=jnp.float32) + vpb_ref[0]

    soT = lax.dot_general(sow_ref[...], query, (((0,), (1,)), ((), ())),
                          preferred_element_type=jnp.float32) + sob_ref[...].reshape(-1, 1)
    awT = lax.dot_general(aww_ref[...], query, (((0,), (1,)), ((), ())),
                          preferred_element_type=jnp.float32) + awb_ref[...].reshape(-1, 1)

    qi = lax.broadcasted_iota(jnp.int32, (1, nq), 1)
    colq = (qi % W).astype(jnp.float32)
    rowq = (qi // W).astype(jnp.float32)

    pad = jnp.zeros((nq, 128 - hd), jnp.float32)
    for h in range(_NH):
        vh_ref[0, h] = jnp.concatenate([v[:, h * hd:(h + 1) * hd], pad], axis=1)
        base = (b * _NH + h) * nq

        rows = [awT[h * _NPNT + p:h * _NPNT + p + 1, :] for p in range(_NPNT)]
        m = jnp.maximum(jnp.maximum(rows[0], rows[1]), jnp.maximum(rows[2], rows[3]))
        es = [jnp.exp(r - m) for r in rows]
        inv = 1.0 / (es[0] + es[1] + es[2] + es[3])

        w_rows = []
        j_rows = []
        for p in range(_NPNT):
            o = (h * _NPNT + p) * 2
            x = colq + soT[o:o + 1, :]
            y = rowq + soT[o + 1:o + 2, :]
            x0f = jnp.floor(x)
            y0f = jnp.floor(y)
            wx1 = x - x0f
            wx0 = 1.0 - wx1
            wy1 = y - y0f
            wy0 = 1.0 - wy1
            x0i = x0f.astype(jnp.int32)
            y0i = y0f.astype(jnp.int32)
            awp = es[p] * inv
            for dy, wyc in ((0, wy0), (1, wy1)):
                yi = y0i + dy
                for dx, wxc in ((0, wx0), (1, wx1)):
                    xi = x0i + dx
                    valid = (xi >= 0) & (xi <= W - 1) & (yi >= 0) & (yi <= W - 1)
                    wc = jnp.where(valid, wxc * wyc * awp, 0.0)
                    jloc = jnp.clip(yi, 0, W - 1) * W + jnp.clip(xi, 0, W - 1)
                    w_rows.append(wc)
                    j_rows.append(jloc + base)
        tw_ref[0, h] = lax.transpose(jnp.concatenate(w_rows, axis=0), (1, 0))
        tidx_ref[0, h] = lax.transpose(jnp.concatenate(j_rows, axis=0), (1, 0))


def _out_body(sam_ref, x2_ref, opw_ref, opb_ref, out_ref):
    nq, C = x2_ref.shape[1], x2_ref.shape[2]
    hd = C // _NH
    sampled = jnp.concatenate([sam_ref[0, h] for h in range(_NH)], axis=1)
    final = jnp.dot(sampled, opw_ref[...], preferred_element_type=jnp.float32)
    out_ref[0] = final + opb_ref[0] + x2_ref[0]


def _sc_sample(table_hbm, tidx_hbm, tw_hbm, out_hbm, idx_v, w_v, rows_v, acc_v, sem):
    nq = out_hbm.shape[1]
    hd = out_hbm.shape[2]
    nchunks = nq // _KQ
    ntap = _KQ * _TAPS
    wid = lax.axis_index("s") * 2 + lax.axis_index("c")
    zeros16 = jnp.zeros((16,), jnp.int32)

    def chunk_body(ci, carry):
        base = ci * _KQ
        pltpu.sync_copy(tidx_hbm.at[wid, pl.ds(base * _TAPS, ntap)], idx_v)
        pltpu.sync_copy(tw_hbm.at[wid, pl.ds(base * _TAPS, ntap)], w_v)
        pltpu.async_copy(table_hbm.at[idx_v], rows_v, sem).wait()

        def q_body(qi, carry2):
            r0 = qi * _TAPS
            accs = [jnp.zeros((16,), jnp.float32) for _ in range(hd // 16)]
            for j in range(_TAPS):
                wj = plsc.load_gather(w_v, [zeros16 + (r0 + j)])
                for cb in range(hd // 16):
                    accs[cb] = accs[cb] + wj * rows_v[r0 + j, pl.ds(cb * 16, 16)]
            for cb in range(hd // 16):
                acc_v[qi, pl.ds(cb * 16, 16)] = accs[cb]
            return carry2

        lax.fori_loop(0, _KQ, q_body, 0)
        pltpu.sync_copy(acc_v, out_hbm.at[wid, pl.ds(base, _KQ)])
        return carry

    lax.fori_loop(0, nchunks, chunk_body, 0)


def kernel(x1, x2, ln1_w, ln1_b, ln2_w, ln2_b, pos_scale, so_w, so_b,
           aw_w, aw_b, vp_w, vp_b, op_w, op_b):
    B, C, H, W = x1.shape
    nq = H * W
    hd = C // _NH

    x1t = x1.reshape(B, C, nq).transpose(0, 2, 1)
    x2t = x2.reshape(B, C, nq).transpose(0, 2, 1)

    inv_freq = 1.0 / (10000.0 ** (jnp.arange(0, C, 2, dtype=jnp.float32) / C))
    t = jnp.arange(nq, dtype=jnp.float32)
    sinu = t[:, None] * inv_freq[None, :]
    qpos = jnp.concatenate([jnp.sin(sinu), jnp.cos(sinu)], axis=-1) * pos_scale

    full = lambda shape: pl.BlockSpec(shape, lambda b: (0,) * len(shape))
    vheads, tidx, tw = pl.pallas_call(
        _proj_body,
        grid=(B,),
        in_specs=[
            pl.BlockSpec((1, nq, C), lambda b: (b, 0, 0)),
            pl.BlockSpec((1, nq, C), lambda b: (b, 0, 0)),
            full((nq, C)),
            full((1, C)), full((1, C)), full((1, C)), full((1, C)),
            full((C, _NH * _NPNT * 2)), full((_NH * _NPNT * 2,)),
            full((C, _NH * _NPNT)), full((_NH * _NPNT,)),
            full((C, C)), full((1, C)),
        ],
        out_specs=[
            pl.BlockSpec((1, _NH, nq, 128), lambda b: (b, 0, 0, 0)),
            pl.BlockSpec((1, _NH, nq, _TAPS), lambda b: (b, 0, 0, 0)),
            pl.BlockSpec((1, _NH, nq, _TAPS), lambda b: (b, 0, 0, 0)),
        ],
        out_shape=[
            jax.ShapeDtypeStruct((B, _NH, nq, 128), jnp.float32),
            jax.ShapeDtypeStruct((B, _NH, nq, _TAPS), jnp.int32),
            jax.ShapeDtypeStruct((B, _NH, nq, _TAPS), jnp.float32),
        ],
    )(x1t, x2t, qpos,
      ln1_w.reshape(1, C), ln1_b.reshape(1, C), ln2_w.reshape(1, C), ln2_b.reshape(1, C),
      so_w, so_b, aw_w, aw_b, vp_w, vp_b.reshape(1, C))

    table = vheads.reshape(B * _NH * nq, 128)
    tidx_f = tidx.reshape(B * _NH, nq * _TAPS)
    tw_f = tw.reshape(B * _NH, nq * _TAPS)

    mesh = plsc.VectorSubcoreMesh(core_axis_name="c", subcore_axis_name="s")
    sampled = pl.kernel(
        _sc_sample,
        out_type=jax.ShapeDtypeStruct((B * _NH, nq, hd), jnp.float32),
        mesh=mesh,
        scratch_types=[
            pltpu.VMEM((_KQ * _TAPS,), jnp.int32),
            pltpu.VMEM((_KQ * _TAPS,), jnp.float32),
            pltpu.VMEM((_KQ * _TAPS, 128), jnp.float32),
            pltpu.VMEM((_KQ, hd), jnp.float32),
            pltpu.SemaphoreType.DMA,
        ],
        compiler_params=pltpu.CompilerParams(needs_layout_passes=False),
    )(table, tidx_f, tw_f)

    sam4 = sampled.reshape(B, _NH, nq, hd)

    out = pl.pallas_call(
        _out_body,
        grid=(B,),
        in_specs=[
            pl.BlockSpec((1, _NH, nq, hd), lambda b: (b, 0, 0, 0)),
            pl.BlockSpec((1, nq, C), lambda b: (b, 0, 0)),
            full((C, C)), full((1, C)),
        ],
        out_specs=pl.BlockSpec((1, nq, C), lambda b: (b, 0, 0)),
        out_shape=jax.ShapeDtypeStruct((B, nq, C), jnp.float32),
    )(sam4, x2t, op_w, op_b.reshape(1, C))
    return out.transpose(0, 2, 1).reshape(B, C, H, W)


# speedup vs baseline: 2.8149x; 2.8149x over previous
import functools

import jax
import jax.numpy as jnp
from jax import lax
from jax.experimental import pallas as pl
from jax.experimental.pallas import tpu as pltpu
from jax.experimental.pallas import tpu_sc as plsc

_NH = 4
_NPNT = 4
_KQ = 8
_TAPS = 16


def _proj_body(x1_ref, x2_ref, qpos_ref, ln1w_ref, ln1b_ref, ln2w_ref, ln2b_ref,
               sow_ref, sob_ref, aww_ref, awb_ref, vpw_ref, vpb_ref,
               vh_ref, tidx_ref, tw_ref):
    nq, C = x1_ref.shape[1], x1_ref.shape[2]
    W = 32
    hd = C // _NH
    b = pl.program_id(0)

    x1b = x1_ref[0]
    x2b = x2_ref[0]

    def ln(x, w, b_):
        mu = jnp.mean(x, axis=-1, keepdims=True)
        xc = x - mu
        var = jnp.mean(xc * xc, axis=-1, keepdims=True)
        return xc * lax.rsqrt(var + 1e-5) * w + b_

    query = ln(x1b, ln1w_ref[0], ln1b_ref[0]) + qpos_ref[...]
    value = ln(x2b, ln2w_ref[0], ln2b_ref[0])
    v = jnp.dot(value, vpw_ref[...], preferred_element_type=jnp.float32) + vpb_ref[0]

    soT = lax.dot_general(sow_ref[...], query, (((0,), (1,)), ((), ())),
                          preferred_element_type=jnp.float32) + sob_ref[...].reshape(-1, 1)
    awT = lax.dot_general(aww_ref[...], query, (((0,), (1,)), ((), ())),
                          preferred_element_type=jnp.float32) + awb_ref[...].reshape(-1, 1)

    qi = lax.broadcasted_iota(jnp.int32, (1, nq), 1)
    colq = (qi % W).astype(jnp.float32)
    rowq = (qi // W).astype(jnp.float32)

    pad = jnp.zeros((nq, 128 - hd), jnp.float32)
    for h in range(_NH):
        vh_ref[0, h] = jnp.concatenate([v[:, h * hd:(h + 1) * hd], pad], axis=1)
        base = (b * _NH + h) * nq

        rows = [awT[h * _NPNT + p:h * _NPNT + p + 1, :] for p in range(_NPNT)]
        m = jnp.maximum(jnp.maximum(rows[0], rows[1]), jnp.maximum(rows[2], rows[3]))
        es = [jnp.exp(r - m) for r in rows]
        inv = 1.0 / (es[0] + es[1] + es[2] + es[3])

        w_rows = []
        j_rows = []
        for p in range(_NPNT):
            o = (h * _NPNT + p) * 2
            x = colq + soT[o:o + 1, :]
            y = rowq + soT[o + 1:o + 2, :]
            x0f = jnp.floor(x)
            y0f = jnp.floor(y)
            wx1 = x - x0f
            wx0 = 1.0 - wx1
            wy1 = y - y0f
            wy0 = 1.0 - wy1
            x0i = x0f.astype(jnp.int32)
            y0i = y0f.astype(jnp.int32)
            awp = es[p] * inv
            for dy, wyc in ((0, wy0), (1, wy1)):
                yi = y0i + dy
                for dx, wxc in ((0, wx0), (1, wx1)):
                    xi = x0i + dx
                    valid = (xi >= 0) & (xi <= W - 1) & (yi >= 0) & (yi <= W - 1)
                    wc = jnp.where(valid, wxc * wyc * awp, 0.0)
                    jloc = jnp.clip(yi, 0, W - 1) * W + jnp.clip(xi, 0, W - 1)
                    w_rows.append(wc)
                    j_rows.append(jloc + base)
        tw_ref[0, h] = lax.transpose(jnp.concatenate(w_rows, axis=0), (1, 0))
        tidx_ref[0, h] = lax.transpose(jnp.concatenate(j_rows, axis=0), (1, 0))


def _dense_body(x1_ref, x2_ref, qpos_ref, ln1w_ref, ln1b_ref, ln2w_ref, ln2b_ref,
                sow_ref, sob_ref, aww_ref, awb_ref, vpw_ref, vpb_ref,
                opw_ref, opb_ref, out_ref):
    nq, C = x1_ref.shape[1], x1_ref.shape[2]
    W = 32
    hd = C // _NH

    x1b = x1_ref[0]
    x2b = x2_ref[0]

    def ln(x, w, b):
        mu = jnp.mean(x, axis=-1, keepdims=True)
        xc = x - mu
        var = jnp.mean(xc * xc, axis=-1, keepdims=True)
        return xc * lax.rsqrt(var + 1e-5) * w + b

    query = ln(x1b, ln1w_ref[0], ln1b_ref[0]) + qpos_ref[...]
    value = ln(x2b, ln2w_ref[0], ln2b_ref[0])
    v = jnp.dot(value, vpw_ref[...], preferred_element_type=jnp.float32) + vpb_ref[0]

    soT = lax.dot_general(sow_ref[...], query, (((0,), (1,)), ((), ())),
                          preferred_element_type=jnp.float32) + sob_ref[...].reshape(-1, 1)
    awT = lax.dot_general(aww_ref[...], query, (((0,), (1,)), ((), ())),
                          preferred_element_type=jnp.float32) + awb_ref[...].reshape(-1, 1)

    qi = lax.broadcasted_iota(jnp.int32, (1, nq), 1)
    colq = (qi % W).astype(jnp.float32)
    rowq = (qi // W).astype(jnp.float32)
    xg = lax.broadcasted_iota(jnp.int32, (W, nq), 0).astype(jnp.float32)

    outs = []
    for h in range(_NH):
        rows = [awT[h * _NPNT + p:h * _NPNT + p + 1, :] for p in range(_NPNT)]
        m = jnp.maximum(jnp.maximum(rows[0], rows[1]), jnp.maximum(rows[2], rows[3]))
        es = [jnp.exp(r - m) for r in rows]
        inv = 1.0 / (es[0] + es[1] + es[2] + es[3])

        at3 = None
        for p in range(_NPNT):
            o = (h * _NPNT + p) * 2
            x = colq + soT[o:o + 1, :]
            y = rowq + soT[o + 1:o + 2, :]
            wx = jnp.maximum(1.0 - jnp.abs(x - xg), 0.0)
            wy = jnp.maximum(1.0 - jnp.abs(y - xg), 0.0)
            wxa = wx * (es[p] * inv)
            term = wy[:, None, :] * wxa[None, :, :]
            at3 = term if at3 is None else at3 + term
        atm = at3.reshape(nq, nq)
        v_h = v[:, h * hd:(h + 1) * hd]
        out_h = lax.dot_general(atm, v_h, (((0,), (0,)), ((), ())),
                                preferred_element_type=jnp.float32)
        outs.append(out_h)

    sampled = jnp.concatenate(outs, axis=1)
    final = jnp.dot(sampled, opw_ref[...], preferred_element_type=jnp.float32)
    out_ref[0] = final + opb_ref[0] + x2b


def _out_body(sam_ref, x2_ref, opw_ref, opb_ref, out_ref):
    nq, C = x2_ref.shape[1], x2_ref.shape[2]
    hd = C // _NH
    sampled = jnp.concatenate([sam_ref[0, h] for h in range(_NH)], axis=1)
    final = jnp.dot(sampled, opw_ref[...], preferred_element_type=jnp.float32)
    out_ref[0] = final + opb_ref[0] + x2_ref[0]


def _sc_sample(table_hbm, tidx_hbm, tw_hbm, out_hbm, idx_v, w_v, rows_v, acc_v, sem):
    nslab = out_hbm.shape[0]
    nq = out_hbm.shape[1]
    hd = out_hbm.shape[2]
    wps = 32 // nslab
    qspan = nq // wps
    nchunks = qspan // _KQ
    ntap = _KQ * _TAPS
    wid = lax.axis_index("s") * 2 + lax.axis_index("c")
    slab = wid // wps
    qoff = (wid % wps) * qspan
    zeros16 = jnp.zeros((16,), jnp.int32)

    def chunk_body(ci, carry):
        base = qoff + ci * _KQ
        pltpu.sync_copy(tidx_hbm.at[slab, pl.ds(base * _TAPS, ntap)], idx_v)
        pltpu.sync_copy(tw_hbm.at[slab, pl.ds(base * _TAPS, ntap)], w_v)
        pltpu.async_copy(table_hbm.at[idx_v], rows_v, sem).wait()

        def q_body(qi, carry2):
            r0 = qi * _TAPS
            accs = [jnp.zeros((16,), jnp.float32) for _ in range(hd // 16)]
            for j in range(_TAPS):
                wj = plsc.load_gather(w_v, [zeros16 + (r0 + j)])
                for cb in range(hd // 16):
                    accs[cb] = accs[cb] + wj * rows_v[r0 + j, pl.ds(cb * 16, 16)]
            for cb in range(hd // 16):
                acc_v[qi, pl.ds(cb * 16, 16)] = accs[cb]
            return carry2

        lax.fori_loop(0, _KQ, q_body, 0)
        pltpu.sync_copy(acc_v, out_hbm.at[slab, pl.ds(base, _KQ)])
        return carry

    lax.fori_loop(0, nchunks, chunk_body, 0)


def kernel(x1, x2, ln1_w, ln1_b, ln2_w, ln2_b, pos_scale, so_w, so_b,
           aw_w, aw_b, vp_w, vp_b, op_w, op_b):
    B, C, H, W = x1.shape
    nq = H * W
    hd = C // _NH
    B_SC = 2
    B_TC = B - B_SC

    x1t = x1.reshape(B, C, nq).transpose(0, 2, 1)
    x2t = x2.reshape(B, C, nq).transpose(0, 2, 1)

    inv_freq = 1.0 / (10000.0 ** (jnp.arange(0, C, 2, dtype=jnp.float32) / C))
    t = jnp.arange(nq, dtype=jnp.float32)
    sinu = t[:, None] * inv_freq[None, :]
    qpos = jnp.concatenate([jnp.sin(sinu), jnp.cos(sinu)], axis=-1) * pos_scale

    full = lambda shape: pl.BlockSpec(shape, lambda b: (0,) * len(shape))

    vheads, tidx, tw = pl.pallas_call(
        _proj_body,
        grid=(B_SC,),
        in_specs=[
            pl.BlockSpec((1, nq, C), lambda b: (b, 0, 0)),
            pl.BlockSpec((1, nq, C), lambda b: (b, 0, 0)),
            full((nq, C)),
            full((1, C)), full((1, C)), full((1, C)), full((1, C)),
            full((C, _NH * _NPNT * 2)), full((_NH * _NPNT * 2,)),
            full((C, _NH * _NPNT)), full((_NH * _NPNT,)),
            full((C, C)), full((1, C)),
        ],
        out_specs=[
            pl.BlockSpec((1, _NH, nq, 128), lambda b: (b, 0, 0, 0)),
            pl.BlockSpec((1, _NH, nq, _TAPS), lambda b: (b, 0, 0, 0)),
            pl.BlockSpec((1, _NH, nq, _TAPS), lambda b: (b, 0, 0, 0)),
        ],
        out_shape=[
            jax.ShapeDtypeStruct((B_SC, _NH, nq, 128), jnp.float32),
            jax.ShapeDtypeStruct((B_SC, _NH, nq, _TAPS), jnp.int32),
            jax.ShapeDtypeStruct((B_SC, _NH, nq, _TAPS), jnp.float32),
        ],
    )(x1t[B_TC:], x2t[B_TC:], qpos,
      ln1_w.reshape(1, C), ln1_b.reshape(1, C), ln2_w.reshape(1, C), ln2_b.reshape(1, C),
      so_w, so_b, aw_w, aw_b, vp_w, vp_b.reshape(1, C))

    table = vheads.reshape(B_SC * _NH * nq, 128)
    tidx_f = tidx.reshape(B_SC * _NH, nq * _TAPS)
    tw_f = tw.reshape(B_SC * _NH, nq * _TAPS)

    mesh = plsc.VectorSubcoreMesh(core_axis_name="c", subcore_axis_name="s")
    sampled = pl.kernel(
        _sc_sample,
        out_type=jax.ShapeDtypeStruct((B_SC * _NH, nq, hd), jnp.float32),
        mesh=mesh,
        scratch_types=[
            pltpu.VMEM((_KQ * _TAPS,), jnp.int32),
            pltpu.VMEM((_KQ * _TAPS,), jnp.float32),
            pltpu.VMEM((_KQ * _TAPS, 128), jnp.float32),
            pltpu.VMEM((_KQ, hd), jnp.float32),
            pltpu.SemaphoreType.DMA,
        ],
        compiler_params=pltpu.CompilerParams(needs_layout_passes=False),
    )(table, tidx_f, tw_f)

    sam4 = sampled.reshape(B_SC, _NH, nq, hd)

    out_sc = pl.pallas_call(
        _out_body,
        grid=(B_SC,),
        in_specs=[
            pl.BlockSpec((1, _NH, nq, hd), lambda b: (b, 0, 0, 0)),
            pl.BlockSpec((1, nq, C), lambda b: (b, 0, 0)),
            full((C, C)), full((1, C)),
        ],
        out_specs=pl.BlockSpec((1, nq, C), lambda b: (b, 0, 0)),
        out_shape=jax.ShapeDtypeStruct((B_SC, nq, C), jnp.float32),
    )(sam4, x2t[B_TC:], op_w, op_b.reshape(1, C))

    out_tc = pl.pallas_call(
        _dense_body,
        grid=(B_TC,),
        in_specs=[
            pl.BlockSpec((1, nq, C), lambda b: (b, 0, 0)),
            pl.BlockSpec((1, nq, C), lambda b: (b, 0, 0)),
            full((nq, C)),
            full((1, C)), full((1, C)), full((1, C)), full((1, C)),
            full((C, _NH * _NPNT * 2)), full((_NH * _NPNT * 2,)),
            full((C, _NH * _NPNT)), full((_NH * _NPNT,)),
            full((C, C)), full((1, C)),
            full((C, C)), full((1, C)),
        ],
        out_specs=pl.BlockSpec((1, nq, C), lambda b: (b, 0, 0)),
        out_shape=jax.ShapeDtypeStruct((B_TC, nq, C), jnp.float32),
    )(x1t[:B_TC], x2t[:B_TC], qpos,
      ln1_w.reshape(1, C), ln1_b.reshape(1, C), ln2_w.reshape(1, C), ln2_b.reshape(1, C),
      so_w, so_b, aw_w, aw_b,
      vp_w, vp_b.reshape(1, C), op_w, op_b.reshape(1, C))

    out = jnp.concatenate([out_tc, out_sc], axis=0)
    return out.transpose(0, 2, 1).reshape(B, C, H, W)


# speedup vs baseline: 4.8103x; 1.7089x over previous
import jax
import jax.numpy as jnp
from jax import lax
from jax.experimental import pallas as pl

_NH = 4
_NPNT = 4


def _body(x1_ref, x2_ref, qpos_ref, ln1w_ref, ln1b_ref, ln2w_ref, ln2b_ref,
          sow_ref, sob_ref, aww_ref, awb_ref, vpw_ref, vpb_ref,
          opw_ref, opb_ref, out_ref):
    nq, C = x1_ref.shape[1], x1_ref.shape[2]
    W = 32
    hd = C // _NH

    x1b = x1_ref[0]
    x2b = x2_ref[0]

    def ln(x, w, b):
        mu = jnp.mean(x, axis=-1, keepdims=True)
        xc = x - mu
        var = jnp.mean(xc * xc, axis=-1, keepdims=True)
        return xc * lax.rsqrt(var + 1e-5) * w + b

    query = ln(x1b, ln1w_ref[0], ln1b_ref[0]) + qpos_ref[...]
    value = ln(x2b, ln2w_ref[0], ln2b_ref[0])

    v = jnp.dot(value, vpw_ref[...], preferred_element_type=jnp.float32) + vpb_ref[0]

    soT = lax.dot_general(sow_ref[...], query, (((0,), (1,)), ((), ())),
                          preferred_element_type=jnp.float32) + sob_ref[...].reshape(-1, 1)
    awT = lax.dot_general(aww_ref[...], query, (((0,), (1,)), ((), ())),
                          preferred_element_type=jnp.float32) + awb_ref[...].reshape(-1, 1)

    qi = lax.broadcasted_iota(jnp.int32, (1, nq), 1)
    colq = (qi % W).astype(jnp.float32)
    rowq = (qi // W).astype(jnp.float32)
    xg = lax.broadcasted_iota(jnp.int32, (W, nq), 0).astype(jnp.float32)

    outs = []
    for h in range(_NH):
        rows = [awT[h * _NPNT + p:h * _NPNT + p + 1, :] for p in range(_NPNT)]
        m = jnp.maximum(jnp.maximum(rows[0], rows[1]), jnp.maximum(rows[2], rows[3]))
        es = [jnp.exp(r - m) for r in rows]
        denom = es[0] + es[1] + es[2] + es[3]
        inv = 1.0 / denom

        at3 = None
        for p in range(_NPNT):
            o = (h * _NPNT + p) * 2
            x = colq + soT[o:o + 1, :]
            y = rowq + soT[o + 1:o + 2, :]
            wx = jnp.maximum(1.0 - jnp.abs(x - xg), 0.0)
            wy = jnp.maximum(1.0 - jnp.abs(y - xg), 0.0)
            wxa = wx * (es[p] * inv)
            term = wy[:, None, :] * wxa[None, :, :]
            at3 = term if at3 is None else at3 + term
        atm = at3.reshape(nq, nq)
        v_h = v[:, h * hd:(h + 1) * hd]
        out_h = lax.dot_general(atm, v_h, (((0,), (0,)), ((), ())),
                                preferred_element_type=jnp.float32)
        outs.append(out_h)

    sampled = jnp.concatenate(outs, axis=1)
    final = jnp.dot(sampled, opw_ref[...], preferred_element_type=jnp.float32)
    out_ref[0] = lax.transpose(final + opb_ref[0] + x2b, (1, 0))


def kernel(x1, x2, ln1_w, ln1_b, ln2_w, ln2_b, pos_scale, so_w, so_b,
           aw_w, aw_b, vp_w, vp_b, op_w, op_b):
    B, C, H, W = x1.shape
    nq = H * W

    x1t = x1.reshape(B, C, nq).transpose(0, 2, 1)
    x2t = x2.reshape(B, C, nq).transpose(0, 2, 1)

    inv_freq = 1.0 / (10000.0 ** (jnp.arange(0, C, 2, dtype=jnp.float32) / C))
    t = jnp.arange(nq, dtype=jnp.float32)
    sinu = t[:, None] * inv_freq[None, :]
    qpos = jnp.concatenate([jnp.sin(sinu), jnp.cos(sinu)], axis=-1) * pos_scale

    full = lambda shape: pl.BlockSpec(shape, lambda b: (0,) * len(shape))
    out = pl.pallas_call(
        _body,
        grid=(B,),
        in_specs=[
            pl.BlockSpec((1, nq, C), lambda b: (b, 0, 0)),
            pl.BlockSpec((1, nq, C), lambda b: (b, 0, 0)),
            full((nq, C)),
            full((1, C)), full((1, C)), full((1, C)), full((1, C)),
            full((C, _NH * _NPNT * 2)), full((_NH * _NPNT * 2,)),
            full((C, _NH * _NPNT)), full((_NH * _NPNT,)),
            full((C, C)), full((1, C)),
            full((C, C)), full((1, C)),
        ],
        out_specs=pl.BlockSpec((1, C, nq), lambda b: (b, 0, 0)),
        out_shape=jax.ShapeDtypeStruct((B, C, nq), jnp.float32),
    )(x1t, x2t, qpos,
      ln1_w.reshape(1, C), ln1_b.reshape(1, C), ln2_w.reshape(1, C), ln2_b.reshape(1, C),
      so_w, so_b, aw_w, aw_b,
      vp_w, vp_b.reshape(1, C), op_w, op_b.reshape(1, C))
    return out.reshape(B, C, H, W)


# speedup vs baseline: 5.5601x; 1.1559x over previous
import jax
import jax.numpy as jnp
from jax import lax
from jax.experimental import pallas as pl

_NH = 4
_NPNT = 4


def _body(x1_ref, x2_ref, qpos_ref, ln1w_ref, ln1b_ref, ln2w_ref, ln2b_ref,
          sow_ref, sob_ref, aww_ref, awb_ref, vpw_ref, vpb_ref,
          opw_ref, opb_ref, out_ref):
    nq, C = x1_ref.shape[1], x1_ref.shape[2]
    W = 32
    hd = C // _NH

    x1b = x1_ref[0]
    x2b = x2_ref[0]

    def ln(x, w, b):
        mu = jnp.mean(x, axis=-1, keepdims=True)
        xc = x - mu
        var = jnp.mean(xc * xc, axis=-1, keepdims=True)
        return xc * lax.rsqrt(var + 1e-5) * w + b

    query = ln(x1b, ln1w_ref[0], ln1b_ref[0]) + qpos_ref[...]
    value = ln(x2b, ln2w_ref[0], ln2b_ref[0])

    v = jnp.dot(value, vpw_ref[...], preferred_element_type=jnp.float32) + vpb_ref[0]

    soT = lax.dot_general(sow_ref[...], query, (((0,), (1,)), ((), ())),
                          preferred_element_type=jnp.float32) + sob_ref[...].reshape(-1, 1)
    awT = lax.dot_general(aww_ref[...], query, (((0,), (1,)), ((), ())),
                          preferred_element_type=jnp.float32) + awb_ref[...].reshape(-1, 1)

    qi = lax.broadcasted_iota(jnp.int32, (1, nq), 1)
    colq = (qi % W).astype(jnp.float32)
    rowq = (qi // W).astype(jnp.float32)
    xg = lax.broadcasted_iota(jnp.int32, (W, nq), 0).astype(jnp.float32)

    outs = []
    for h in range(_NH):
        rows = [awT[h * _NPNT + p:h * _NPNT + p + 1, :] for p in range(_NPNT)]
        m = jnp.maximum(jnp.maximum(rows[0], rows[1]), jnp.maximum(rows[2], rows[3]))
        es = [jnp.exp(r - m) for r in rows]
        denom = es[0] + es[1] + es[2] + es[3]
        inv = 1.0 / denom

        at3 = None
        for p in range(_NPNT):
            o = (h * _NPNT + p) * 2
            x = colq + soT[o:o + 1, :]
            y = rowq + soT[o + 1:o + 2, :]
            wx = jnp.maximum(1.0 - jnp.abs(x - xg), 0.0)
            wy = jnp.maximum(1.0 - jnp.abs(y - xg), 0.0)
            wxa = wx * (es[p] * inv)
            term = wy[:, None, :] * wxa[None, :, :]
            at3 = term if at3 is None else at3 + term
        atm = at3.reshape(nq, nq)
        v_h = v[:, h * hd:(h + 1) * hd]
        out_h = lax.dot_general(atm, v_h, (((0,), (0,)), ((), ())),
                                preferred_element_type=jnp.float32)
        outs.append(out_h)

    sampled = jnp.concatenate(outs, axis=1)
    final = jnp.dot(sampled, opw_ref[...], preferred_element_type=jnp.float32)
    out_ref[0] = final + opb_ref[0] + x2b


def kernel(x1, x2, ln1_w, ln1_b, ln2_w, ln2_b, pos_scale, so_w, so_b,
           aw_w, aw_b, vp_w, vp_b, op_w, op_b):
    B, C, H, W = x1.shape
    nq = H * W

    x1t = x1.reshape(B, C, nq).transpose(0, 2, 1)
    x2t = x2.reshape(B, C, nq).transpose(0, 2, 1)

    inv_freq = 1.0 / (10000.0 ** (jnp.arange(0, C, 2, dtype=jnp.float32) / C))
    t = jnp.arange(nq, dtype=jnp.float32)
    sinu = t[:, None] * inv_freq[None, :]
    qpos = jnp.concatenate([jnp.sin(sinu), jnp.cos(sinu)], axis=-1) * pos_scale

    full = lambda shape: pl.BlockSpec(shape, lambda b: (0,) * len(shape))
    out = pl.pallas_call(
        _body,
        grid=(B,),
        in_specs=[
            pl.BlockSpec((1, nq, C), lambda b: (b, 0, 0)),
            pl.BlockSpec((1, nq, C), lambda b: (b, 0, 0)),
            full((nq, C)),
            full((1, C)), full((1, C)), full((1, C)), full((1, C)),
            full((C, _NH * _NPNT * 2)), full((_NH * _NPNT * 2,)),
            full((C, _NH * _NPNT)), full((_NH * _NPNT,)),
            full((C, C)), full((1, C)),
            full((C, C)), full((1, C)),
        ],
        out_specs=pl.BlockSpec((1, nq, C), lambda b: (b, 0, 0)),
        out_shape=jax.ShapeDtypeStruct((B, nq, C), jnp.float32),
    )(x1t, x2t, qpos,
      ln1_w.reshape(1, C), ln1_b.reshape(1, C), ln2_w.reshape(1, C), ln2_b.reshape(1, C),
      so_w, so_b, aw_w, aw_b,
      vp_w, vp_b.reshape(1, C), op_w, op_b.reshape(1, C))
    return out.transpose(0, 2, 1).reshape(B, C, H, W)
